# Initial kernel scaffold; baseline (speedup 1.0000x reference)
import jax
import jax.numpy as jnp
from jax.experimental import pallas as pl


def kernel(x, qkv_w, offset_w, proj_w, proj_b):
    raise NotImplementedError("write your pallas kernel here")


# speedup vs baseline: 1.0000x; 1.0000x over previous
import jax, jax.numpy as jnp
from jax.experimental import pallas as pl

_NUM_HEADS = 8
_NS = 9


def _deform_unfold(img, offset):
    BH, Cd, H, W = img.shape
    ky = (jnp.repeat(jnp.arange(3), 3) - 1).astype(jnp.float32)
    kx = (jnp.tile(jnp.arange(3), 3) - 1).astype(jnp.float32)
    gy = jnp.arange(H, dtype=jnp.float32).reshape(1, 1, H, 1)
    gx = jnp.arange(W, dtype=jnp.float32).reshape(1, 1, 1, W)
    oy = offset[:, 0::2]
    ox = offset[:, 1::2]
    py = gy + ky.reshape(1, _NS, 1, 1) + oy
    px = gx + kx.reshape(1, _NS, 1, 1) + ox
    y0 = jnp.floor(py)
    x0 = jnp.floor(px)
    flat = img.reshape(BH, Cd, H * W)

    def gather(yi, xi):
        valid = (yi >= 0) & (yi <= H - 1) & (xi >= 0) & (xi <= W - 1)
        yc = jnp.clip(yi, 0, H - 1).astype(jnp.int32)
        xc = jnp.clip(xi, 0, W - 1).astype(jnp.int32)
        idx = (yc * W + xc).reshape(BH, -1)
        g = jax.vmap(lambda f, i: jnp.take(f, i, axis=1))(flat, idx)
        g = g.reshape(BH, Cd, _NS, H, W)
        return g * valid.reshape(BH, 1, _NS, H, W).astype(img.dtype)

    wy1 = py - y0
    wy0 = 1.0 - wy1
    wx1 = px - x0
    wx0 = 1.0 - wx1
    out = jnp.zeros((BH, Cd, _NS, H, W), dtype=img.dtype)
    for yi, xi, wgt in ((y0, x0, wy0 * wx0), (y0, x0 + 1.0, wy0 * wx1), (y0 + 1.0, x0, wy1 * wx0), (y0 + 1.0, x0 + 1.0, wy1 * wx1)):
        out = out + gather(yi, xi) * wgt.reshape(BH, 1, _NS, H, W)
    return out.reshape(BH, Cd * _NS, H * W)


def kernel(x, qkv_w, offset_w, proj_w, proj_b):
    B, C, W, H = x.shape
    num_heads = _NUM_HEADS
    head_dim = C // num_heads
    scale = head_dim ** (-0.5)
    ns = _NS
    N = W * H
    xf = jnp.transpose(x, (0, 2, 3, 1)).reshape(B, N, C)
    qkv = (xf @ qkv_w.T).reshape(B, N, 3, num_heads, head_dim)
    qkv = jnp.transpose(qkv, (2, 0, 3, 1, 4))
    q, k, v = qkv[0], qkv[1], qkv[2]
    off = xf.reshape(B, N, num_heads, head_dim) @ offset_w.T
    off = jnp.transpose(off, (0, 2, 3, 1)).reshape(B * num_heads, ns * 2, H, W)
    k4 = jnp.transpose(k, (0, 1, 3, 2)).reshape(B * num_heads, head_dim, H, W)
    v4 = jnp.transpose(v, (0, 1, 3, 2)).reshape(B * num_heads, head_dim, H, W)
    ku = _deform_unfold(k4, off)
    ku = jnp.transpose(ku, (0, 2, 1)).reshape(B, num_heads, N, head_dim, ns)
    vu = _deform_unfold(v4, off).reshape(B, num_heads, head_dim, ns, N)
    vu = jnp.transpose(vu, (0, 1, 4, 3, 2))
    attn = jnp.matmul(q[:, :, :, None, :], ku) * scale
    attn = jax.nn.softmax(attn, axis=-1)
    out = jnp.matmul(attn, vu)
    out = jnp.transpose(out, (0, 2, 1, 3, 4)).reshape(B, N, C)
    out = out @ proj_w.T + proj_b
    out = out.reshape(B, W, H, C)
    return jnp.transpose(out, (0, 3, 1, 2))


# speedup vs baseline: 108.7902x; 108.7886x over previous
import functools
import jax
import jax.numpy as jnp
from jax import lax
from jax.experimental import pallas as pl
from jax.experimental.pallas import tpu as pltpu
from jax.experimental.pallas import tpu_sc as plsc

_B, _C, _WH = 4, 192, 56
_HEADS, _HD, _NS = 8, 24, 9
_N = _WH * _WH
_BH = _B * _HEADS
_P = 224
_NCH = _N // _P
_SCALE = _HD ** -0.5
_MB = 1568


def _mm_kernel(a_ref, b_ref, o_ref):
    o_ref[...] = jnp.dot(a_ref[...], b_ref[...], preferred_element_type=jnp.float32)


def _mm(a, b):
    m, k = a.shape
    _, n = b.shape
    return pl.pallas_call(
        _mm_kernel,
        grid=(m // _MB,),
        in_specs=[
            pl.BlockSpec((_MB, k), lambda i: (i, 0)),
            pl.BlockSpec((k, n), lambda i: (0, 0)),
        ],
        out_specs=pl.BlockSpec((_MB, n), lambda i: (i, 0)),
        out_shape=jax.ShapeDtypeStruct((m, n), jnp.float32),
    )(a, b)


def _mm_bias_kernel(a_ref, b_ref, bias_ref, o_ref):
    o_ref[...] = (
        jnp.dot(a_ref[...], b_ref[...], preferred_element_type=jnp.float32)
        + bias_ref[...]
    )


def _mm_bias(a, b, bias):
    m, k = a.shape
    _, n = b.shape
    return pl.pallas_call(
        _mm_bias_kernel,
        grid=(m // _MB,),
        in_specs=[
            pl.BlockSpec((_MB, k), lambda i: (i, 0)),
            pl.BlockSpec((k, n), lambda i: (0, 0)),
            pl.BlockSpec((1, n), lambda i: (0, 0)),
        ],
        out_specs=pl.BlockSpec((_MB, n), lambda i: (i, 0)),
        out_shape=jax.ShapeDtypeStruct((m, n), jnp.float32),
    )(a, b, bias)


def _idxw_kernel(oy_ref, ox_ref, idx_ref, w_ref):
    jc = pl.program_id(1)
    oy = oy_ref[0, 0]
    ox = ox_ref[0, 0]
    s = lax.broadcasted_iota(jnp.int32, (_NS, _P), 0)
    n = jc * _P + lax.broadcasted_iota(jnp.int32, (_NS, _P), 1)
    rowf = (n // _WH).astype(jnp.float32)
    colf = (n % _WH).astype(jnp.float32)
    ky = (s // 3 - 1).astype(jnp.float32)
    kx = (s % 3 - 1).astype(jnp.float32)
    py = rowf + ky + oy
    px = colf + kx + ox
    y0 = jnp.floor(py)
    x0 = jnp.floor(px)
    wy1 = py - y0
    wy0 = 1.0 - wy1
    wx1 = px - x0
    wx0 = 1.0 - wx1
    idx_parts = []
    w_parts = []
    lim = float(_WH - 1)
    for yi, xi, wgt in (
        (y0, x0, wy0 * wx0),
        (y0, x0 + 1.0, wy0 * wx1),
        (y0 + 1.0, x0, wy1 * wx0),
        (y0 + 1.0, x0 + 1.0, wy1 * wx1),
    ):
        valid = (yi >= 0) & (yi <= lim) & (xi >= 0) & (xi <= lim)
        yc = jnp.clip(yi, 0.0, lim).astype(jnp.int32)
        xc = jnp.clip(xi, 0.0, lim).astype(jnp.int32)
        idx_parts.append(yc * _WH + xc)
        w_parts.append(wgt * valid.astype(jnp.float32))
    idx_ref[0, 0] = jnp.concatenate(idx_parts, axis=0)
    w_ref[0, 0] = jnp.concatenate(w_parts, axis=0)


def _idxw(oy, ox):
    return pl.pallas_call(
        _idxw_kernel,
        grid=(_BH, _NCH),
        in_specs=[
            pl.BlockSpec((1, 1, _NS, _P), lambda b, j: (b, j, 0, 0)),
            pl.BlockSpec((1, 1, _NS, _P), lambda b, j: (b, j, 0, 0)),
        ],
        out_specs=[
            pl.BlockSpec((1, 1, 4 * _NS, _P), lambda b, j: (b, j, 0, 0)),
            pl.BlockSpec((1, 1, 4 * _NS, _P), lambda b, j: (b, j, 0, 0)),
        ],
        out_shape=[
            jax.ShapeDtypeStruct((_BH, _NCH, 4 * _NS, _P), jnp.int32),
            jax.ShapeDtypeStruct((_BH, _NCH, 4 * _NS, _P), jnp.float32),
        ],
    )(oy, ox)


_SC_MESH = plsc.VectorSubcoreMesh(core_axis_name="c", subcore_axis_name="s")
_SC_PARAMS = pltpu.CompilerParams(needs_layout_passes=False)


def _wid():
    return lax.axis_index("s") * 2 + lax.axis_index("c")


@functools.partial(
    pl.kernel,
    out_type=jax.ShapeDtypeStruct((_BH, _NCH, _NS, _P), jnp.float32),
    mesh=_SC_MESH,
    compiler_params=_SC_PARAMS,
    scratch_types=[
        pltpu.VMEM((_HD, _N), jnp.float32),
        pltpu.VMEM((_HD, _P), jnp.float32),
        pltpu.VMEM((4 * _NS, _P), jnp.int32),
        pltpu.VMEM((4 * _NS, _P), jnp.float32),
        pltpu.VMEM((_NS, _P), jnp.float32),
    ],
)
def _sc_logits(ktab_h, q_h, idx_h, w_h, p_h, ktab, qb, ib, wb, pb):
    wid = _wid()
    pltpu.sync_copy(ktab_h.at[wid], ktab)

    def chunk(jc, carry):
        pltpu.sync_copy(q_h.at[wid, jc], qb)
        pltpu.sync_copy(idx_h.at[wid, jc], ib)
        pltpu.sync_copy(w_h.at[wid, jc], wb)

        def tile(t, carry2):
            sl = pl.ds(t * 16, 16)
            qv = [qb[d, sl] for d in range(_HD)]
            logits = []
            for s in range(_NS):
                acc = None
                for c in range(4):
                    iv = ib[c * _NS + s, sl]
                    wv = wb[c * _NS + s, sl]
                    dot = None
                    for d in range(_HD):
                        g = plsc.load_gather(
                            ktab, [jnp.full((16,), d, jnp.int32), iv]
                        )
                        term = qv[d] * g
                        dot = term if dot is None else dot + term
                    acc = wv * dot if acc is None else acc + wv * dot
                logits.append(acc * _SCALE)
            m = logits[0]
            for s in range(1, _NS):
                m = jnp.maximum(m, logits[s])
            es = [jnp.exp(l - m) for l in logits]
            tot = es[0]
            for s in range(1, _NS):
                tot = tot + es[s]
            for s in range(_NS):
                pb[s, sl] = es[s] / tot
            return carry2

        lax.fori_loop(0, _P // 16, tile, 0)
        pltpu.sync_copy(pb, p_h.at[wid, jc])
        return carry

    lax.fori_loop(0, _NCH, chunk, 0)


@functools.partial(
    pl.kernel,
    out_type=jax.ShapeDtypeStruct((_BH, _NCH, _HD, _P), jnp.float32),
    mesh=_SC_MESH,
    compiler_params=_SC_PARAMS,
    scratch_types=[
        pltpu.VMEM((_HD, _N), jnp.float32),
        pltpu.VMEM((_NS, _P), jnp.float32),
        pltpu.VMEM((4 * _NS, _P), jnp.int32),
        pltpu.VMEM((4 * _NS, _P), jnp.float32),
        pltpu.VMEM((_HD, _P), jnp.float32),
    ],
)
def _sc_out(vtab_h, p_h, idx_h, w_h, o_h, vtab, pbuf, ib, wb, ob):
    wid = _wid()
    pltpu.sync_copy(vtab_h.at[wid], vtab)

    def chunk(jc, carry):
        pltpu.sync_copy(p_h.at[wid, jc], pbuf)
        pltpu.sync_copy(idx_h.at[wid, jc], ib)
        pltpu.sync_copy(w_h.at[wid, jc], wb)

        def tile(t, carry2):
            sl = pl.ds(t * 16, 16)
            pv = [pbuf[s, sl] for s in range(_NS)]
            outs = [None] * _HD
            for s in range(_NS):
                for c in range(4):
                    iv = ib[c * _NS + s, sl]
                    wp = wb[c * _NS + s, sl] * pv[s]
                    for d in range(_HD):
                        g = plsc.load_gather(
                            vtab, [jnp.full((16,), d, jnp.int32), iv]
                        )
                        term = wp * g
                        outs[d] = term if outs[d] is None else outs[d] + term
            for d in range(_HD):
                ob[d, sl] = outs[d]
            return carry2

        lax.fori_loop(0, _P // 16, tile, 0)
        pltpu.sync_copy(ob, o_h.at[wid, jc])
        return carry

    lax.fori_loop(0, _NCH, chunk, 0)


def kernel(x, qkv_w, offset_w, proj_w, proj_b):
    B, C, W, H = x.shape
    heads, hd, ns = _HEADS, _HD, _NS
    N = W * H

    blk = jnp.zeros((heads, hd, heads, 2 * ns), jnp.float32)
    eye = jnp.eye(heads, dtype=jnp.float32)
    blk = eye[:, None, :, None] * offset_w.T[None, :, None, :]
    blk = blk.reshape(C, heads * 2 * ns)
    wcat = jnp.concatenate([qkv_w.T, blk], axis=1)

    xf = jnp.transpose(x, (0, 2, 3, 1)).reshape(B * N, C)
    y = _mm(xf, wcat)

    q = y[:, 0:C].reshape(B, N, heads, hd)
    k = y[:, C:2 * C].reshape(B, N, heads, hd)
    v = y[:, 2 * C:3 * C].reshape(B, N, heads, hd)
    off = y[:, 3 * C:].reshape(B, N, heads, 2 * ns)

    ktab = jnp.transpose(k, (0, 2, 3, 1)).reshape(_BH, hd, N)
    vtab = jnp.transpose(v, (0, 2, 3, 1)).reshape(_BH, hd, N)
    q_cm = jnp.transpose(
        q.reshape(B, _NCH, _P, heads, hd), (0, 3, 1, 4, 2)
    ).reshape(_BH, _NCH, hd, _P)
    oy = jnp.transpose(off[..., 0::2], (0, 2, 3, 1)).reshape(_BH, ns, _NCH, _P)
    ox = jnp.transpose(off[..., 1::2], (0, 2, 3, 1)).reshape(_BH, ns, _NCH, _P)
    oy = jnp.transpose(oy, (0, 2, 1, 3))
    ox = jnp.transpose(ox, (0, 2, 1, 3))

    idx_cm, w_cm = _idxw(oy, ox)
    p_cm = _sc_logits(ktab, q_cm, idx_cm, w_cm)
    out_cm = _sc_out(vtab, p_cm, idx_cm, w_cm)

    out_f = jnp.transpose(
        out_cm.reshape(B, heads, _NCH, hd, _P), (0, 2, 4, 1, 3)
    ).reshape(B * N, C)
    fin = _mm_bias(out_f, proj_w.T, proj_b.reshape(1, C))
    fin = fin.reshape(B, W, H, C)
    return jnp.transpose(fin, (0, 3, 1, 2))
